# Initial kernel scaffold; baseline (speedup 1.0000x reference)
import jax
import jax.numpy as jnp
from jax.experimental import pallas as pl


def kernel(x, edge_index, edge_attr, atom_tables, bond0, bond1, bond2, W1, b1, g1, be1, W2, b2, eps, og, ob):
    raise NotImplementedError("write your pallas kernel here")


# speedup vs baseline: 2.0931x; 2.0931x over previous
import functools

import jax
import jax.numpy as jnp
from jax import lax
from jax.experimental import pallas as pl
from jax.experimental.pallas import tpu as pltpu
from jax.experimental.pallas import tpu_sc as plsc

N = 10000
E = 160000
EMB = 256
L = 5

NC = 2
NS = 16
HALF = EMB // NC
LANES = 16
CHUNK = 128
NCHUNK = E // CHUNK
SC_ITERS = -(-NCHUNK // NS)
ZR = 624
ZTAIL = N - ZR * NS


def _sc_msg_body(h2, gidx, gcode, dstl, ee2, agg_out,
                 idx_v, code_v, dst_v, rows_v, aggs, sem):
    c = lax.axis_index("c")
    s = lax.axis_index("s")

    @pl.loop(0, CHUNK)
    def _(r):
        for j in range(HALF // LANES):
            rows_v[r, pl.ds(LANES * j, LANES)] = jnp.zeros((LANES,), jnp.float32)

    base = ZR * s
    nfull = ZR // CHUNK
    for k in range(nfull):
        pltpu.sync_copy(rows_v, aggs.at[pl.ds(base + k * CHUNK, CHUNK)])
    rem = ZR - nfull * CHUNK
    if rem:
        pltpu.sync_copy(rows_v.at[pl.ds(0, rem)],
                        aggs.at[pl.ds(base + nfull * CHUNK, rem)])

    @pl.when(s == NS - 1)
    def _():
        pltpu.sync_copy(rows_v.at[pl.ds(0, ZTAIL)],
                        aggs.at[pl.ds(ZR * NS, ZTAIL)])

    plsc.subcore_barrier()

    @pl.loop(0, SC_ITERS)
    def _(i):
        cid = i * NS + s

        @pl.when(cid < NCHUNK)
        def _():
            off = cid * CHUNK
            pltpu.sync_copy(gidx.at[c, 0, pl.ds(off, CHUNK)], idx_v)
            pltpu.sync_copy(gcode.at[c, 0, pl.ds(off, CHUNK)], code_v)
            pltpu.sync_copy(dstl.at[pl.ds(off, CHUNK)], dst_v)
            pltpu.async_copy(h2.at[idx_v], rows_v, sem).wait()
            pltpu.async_copy(ee2.at[code_v], rows_v, sem, add=True).wait()

            @pl.loop(0, CHUNK)
            def _(r):
                for j in range(HALF // LANES):
                    sl = pl.ds(LANES * j, LANES)
                    rows_v[r, sl] = jnp.maximum(rows_v[r, sl], 0.0)

            pltpu.sync_copy(rows_v, aggs.at[dst_v], add=True)

    plsc.subcore_barrier()
    pltpu.sync_copy(aggs.at[pl.ds(ZR * s, ZR)],
                    agg_out.at[c, pl.ds(ZR * s, ZR)])

    @pl.when(s == NS - 1)
    def _():
        pltpu.sync_copy(aggs.at[pl.ds(ZR * NS, ZTAIL)],
                        agg_out.at[c, pl.ds(ZR * NS, ZTAIL)])


_sc_msg = pl.kernel(
    _sc_msg_body,
    out_type=jax.ShapeDtypeStruct((NC, N, HALF), jnp.float32),
    mesh=plsc.VectorSubcoreMesh(core_axis_name="c", subcore_axis_name="s",
                                num_cores=NC, num_subcores=NS),
    scratch_types=[
        pltpu.VMEM((CHUNK,), jnp.int32),
        pltpu.VMEM((CHUNK,), jnp.int32),
        pltpu.VMEM((CHUNK,), jnp.int32),
        pltpu.VMEM((CHUNK, HALF), jnp.float32),
        pltpu.VMEM_SHARED((N, HALF), jnp.float32),
        pltpu.SemaphoreType.DMA,
    ],
)


def _prep_body(ei_ref, eaT_ref, gidx_ref, gcode_ref):
    src = ei_ref[0:1, :]
    code = eaT_ref[0:1, :] + 2 * eaT_ref[1:2, :] + 4 * eaT_ref[2:3, :]
    gidx_ref[0] = 2 * src
    gidx_ref[1] = 2 * src + 1
    gcode_ref[0] = 2 * code
    gcode_ref[1] = 2 * code + 1


_prep = pl.pallas_call(
    _prep_body,
    out_shape=(jax.ShapeDtypeStruct((2, 1, E), jnp.int32),
               jax.ShapeDtypeStruct((2, 1, E), jnp.int32)),
)


def _atom_body(x_ref, d_ref, base_ref, h_ref):
    h_ref[...] = (jnp.dot(x_ref[...].astype(jnp.float32), d_ref[...],
                          preferred_element_type=jnp.float32,
                          precision=lax.Precision.HIGHEST)
                  + base_ref[...])


_atom = pl.pallas_call(
    _atom_body,
    out_shape=jax.ShapeDtypeStruct((N, EMB), jnp.float32),
)


BN = 2000
NB = N // BN


def _accum_stats(st_ref, y):
    i = pl.program_id(0)
    st = jnp.concatenate([jnp.sum(y, axis=0, keepdims=True),
                          jnp.sum(y * y, axis=0, keepdims=True)], axis=0)

    @pl.when(i == 0)
    def _():
        st_ref[...] = st

    @pl.when(i > 0)
    def _():
        st_ref[...] = st_ref[...] + st


def _mlp1_body(h_ref, a0_ref, a1_ref, sc_ref, W1_ref, b1_ref, y_ref, st_ref):
    agg = jnp.concatenate([a0_ref[...], a1_ref[...]], axis=1)
    z = sc_ref[0, 0] * h_ref[...] + agg
    y = jnp.dot(z, W1_ref[...], preferred_element_type=jnp.float32) + b1_ref[...]
    y_ref[...] = y
    _accum_stats(st_ref, y)


_mlp1 = pl.pallas_call(
    _mlp1_body,
    grid=(NB,),
    in_specs=[
        pl.BlockSpec((BN, EMB), lambda i: (i, 0)),
        pl.BlockSpec((BN, HALF), lambda i: (i, 0)),
        pl.BlockSpec((BN, HALF), lambda i: (i, 0)),
        pl.BlockSpec((1, 1), lambda i: (0, 0)),
        pl.BlockSpec((EMB, 2 * EMB), lambda i: (0, 0)),
        pl.BlockSpec((1, 2 * EMB), lambda i: (0, 0)),
    ],
    out_specs=(pl.BlockSpec((BN, 2 * EMB), lambda i: (i, 0)),
               pl.BlockSpec((2, 2 * EMB), lambda i: (0, 0))),
    out_shape=(jax.ShapeDtypeStruct((N, 2 * EMB), jnp.float32),
               jax.ShapeDtypeStruct((2, 2 * EMB), jnp.float32)),
)


def _mlp2_body(y_ref, st_ref, g1_ref, be1_ref, W2_ref, b2_ref,
               z_ref, st2_ref):
    mu = st_ref[0:1, :] * (1.0 / N)
    var = st_ref[1:2, :] * (1.0 / N) - mu * mu
    yn = (y_ref[...] - mu) * lax.rsqrt(var + 1e-5) * g1_ref[...] + be1_ref[...]
    yn = jnp.maximum(yn, 0.0)
    z2 = jnp.dot(yn, W2_ref[...], preferred_element_type=jnp.float32) + b2_ref[...]
    z_ref[...] = z2
    _accum_stats(st2_ref, z2)


_mlp2 = pl.pallas_call(
    _mlp2_body,
    grid=(NB,),
    in_specs=[
        pl.BlockSpec((BN, 2 * EMB), lambda i: (i, 0)),
        pl.BlockSpec((2, 2 * EMB), lambda i: (0, 0)),
        pl.BlockSpec((1, 2 * EMB), lambda i: (0, 0)),
        pl.BlockSpec((1, 2 * EMB), lambda i: (0, 0)),
        pl.BlockSpec((2 * EMB, EMB), lambda i: (0, 0)),
        pl.BlockSpec((1, EMB), lambda i: (0, 0)),
    ],
    out_specs=(pl.BlockSpec((BN, EMB), lambda i: (i, 0)),
               pl.BlockSpec((2, EMB), lambda i: (0, 0))),
    out_shape=(jax.ShapeDtypeStruct((N, EMB), jnp.float32),
               jax.ShapeDtypeStruct((2, EMB), jnp.float32)),
)


def _mlp3_body(z_ref, st2_ref, og_ref, ob_ref, out_ref, *, last):
    mu = st2_ref[0:1, :] * (1.0 / N)
    var = st2_ref[1:2, :] * (1.0 / N) - mu * mu
    o = (z_ref[...] - mu) * lax.rsqrt(var + 1e-5) * og_ref[...] + ob_ref[...]
    if not last:
        o = jnp.maximum(o, 0.0)
    out_ref[...] = o


def _make_mlp3(last):
    return pl.pallas_call(
        functools.partial(_mlp3_body, last=last),
        grid=(NB,),
        in_specs=[
            pl.BlockSpec((BN, EMB), lambda i: (i, 0)),
            pl.BlockSpec((2, EMB), lambda i: (0, 0)),
            pl.BlockSpec((1, EMB), lambda i: (0, 0)),
            pl.BlockSpec((1, EMB), lambda i: (0, 0)),
        ],
        out_specs=pl.BlockSpec((BN, EMB), lambda i: (i, 0)),
        out_shape=jax.ShapeDtypeStruct((N, EMB), jnp.float32),
    )


_mlp3_mid = _make_mlp3(False)
_mlp3_last = _make_mlp3(True)


def kernel(x, edge_index, edge_attr, atom_tables, bond0, bond1, bond2,
           W1, b1, g1, be1, W2, b2, eps, og, ob):
    f32 = jnp.float32
    d_atom = atom_tables[:, 1, :] - atom_tables[:, 0, :]
    base_atom = atom_tables[:, 0, :].sum(0)[None].astype(f32)
    d16 = jnp.zeros((16, EMB), f32).at[:9].set(d_atom)
    x16 = jnp.zeros((N, 16), jnp.int32).at[:, :9].set(x)
    q = jnp.arange(8)
    ee8 = bond0[:, q & 1] + bond1[:, (q >> 1) & 1] + bond2[:, (q >> 2) & 1]
    ee2 = ee8.reshape(L, 2 * 8, HALF)

    gidx, gcode = _prep(edge_index, edge_attr.T)
    dstl = edge_index[1]

    h = _atom(x16, d16, base_atom)
    for l in range(L):
        agg2 = _sc_msg(h.reshape(2 * N, HALF), gidx, gcode, dstl, ee2[l])
        sc = jnp.full((1, 1), 1.0, f32) + eps[l]
        y, st = _mlp1(h, agg2[0], agg2[1], sc, W1[l], b1[l][None])
        z2, st2 = _mlp2(y, st, g1[l][None], be1[l][None], W2[l], b2[l][None])
        mlp3 = _mlp3_last if l == L - 1 else _mlp3_mid
        h = mlp3(z2, st2, og[l][None], ob[l][None])
    return h


# speedup vs baseline: 2.0940x; 1.0004x over previous
import functools

import jax
import jax.numpy as jnp
from jax import lax
from jax.experimental import pallas as pl
from jax.experimental.pallas import tpu as pltpu
from jax.experimental.pallas import tpu_sc as plsc

N = 10000
E = 160000
EMB = 256
L = 5

NC = 2
NS = 16
HALF = EMB // NC
LANES = 16
CHUNK = 128
NCHUNK = E // CHUNK
SC_ITERS = -(-NCHUNK // NS)
ZR = 624
ZTAIL = N - ZR * NS


def _sc_msg_body(h2, gidx, gcode, dstl, ee2, agg_out,
                 idx_v, code_v, dst_v, rows_v, aggs, sem):
    c = lax.axis_index("c")
    s = lax.axis_index("s")

    @pl.loop(0, CHUNK)
    def _(r):
        for j in range(HALF // LANES):
            rows_v[r, pl.ds(LANES * j, LANES)] = jnp.zeros((LANES,), jnp.float32)

    base = ZR * s
    nfull = ZR // CHUNK
    for k in range(nfull):
        pltpu.sync_copy(rows_v, aggs.at[pl.ds(base + k * CHUNK, CHUNK)])
    rem = ZR - nfull * CHUNK
    if rem:
        pltpu.sync_copy(rows_v.at[pl.ds(0, rem)],
                        aggs.at[pl.ds(base + nfull * CHUNK, rem)])

    @pl.when(s == NS - 1)
    def _():
        pltpu.sync_copy(rows_v.at[pl.ds(0, ZTAIL)],
                        aggs.at[pl.ds(ZR * NS, ZTAIL)])

    plsc.subcore_barrier()

    @pl.loop(0, SC_ITERS)
    def _(i):
        cid = i * NS + s

        @pl.when(cid < NCHUNK)
        def _():
            off = cid * CHUNK
            pltpu.sync_copy(gidx.at[c, 0, pl.ds(off, CHUNK)], idx_v)
            pltpu.sync_copy(gcode.at[c, 0, pl.ds(off, CHUNK)], code_v)
            pltpu.sync_copy(dstl.at[pl.ds(off, CHUNK)], dst_v)
            pltpu.async_copy(h2.at[idx_v], rows_v, sem).wait()
            pltpu.async_copy(ee2.at[code_v], rows_v, sem, add=True).wait()

            @pl.loop(0, CHUNK)
            def _(r):
                for j in range(HALF // LANES):
                    sl = pl.ds(LANES * j, LANES)
                    rows_v[r, sl] = jnp.maximum(rows_v[r, sl], 0.0)

            pltpu.sync_copy(rows_v, aggs.at[dst_v], add=True)

    plsc.subcore_barrier()
    pltpu.sync_copy(aggs.at[pl.ds(ZR * s, ZR)],
                    agg_out.at[c, pl.ds(ZR * s, ZR)])

    @pl.when(s == NS - 1)
    def _():
        pltpu.sync_copy(aggs.at[pl.ds(ZR * NS, ZTAIL)],
                        agg_out.at[c, pl.ds(ZR * NS, ZTAIL)])


_sc_msg = pl.kernel(
    _sc_msg_body,
    out_type=jax.ShapeDtypeStruct((NC, N, HALF), jnp.float32),
    mesh=plsc.VectorSubcoreMesh(core_axis_name="c", subcore_axis_name="s",
                                num_cores=NC, num_subcores=NS),
    scratch_types=[
        pltpu.VMEM((CHUNK,), jnp.int32),
        pltpu.VMEM((CHUNK,), jnp.int32),
        pltpu.VMEM((CHUNK,), jnp.int32),
        pltpu.VMEM((CHUNK, HALF), jnp.float32),
        pltpu.VMEM_SHARED((N, HALF), jnp.float32),
        pltpu.SemaphoreType.DMA,
    ],
)


def _prep_body(ei_ref, eaT_ref, gidx_ref, gcode_ref):
    src = ei_ref[0:1, :]
    code = eaT_ref[0:1, :] + 2 * eaT_ref[1:2, :] + 4 * eaT_ref[2:3, :]
    gidx_ref[0] = 2 * src
    gidx_ref[1] = 2 * src + 1
    gcode_ref[0] = 2 * code
    gcode_ref[1] = 2 * code + 1


_prep = pl.pallas_call(
    _prep_body,
    out_shape=(jax.ShapeDtypeStruct((2, 1, E), jnp.int32),
               jax.ShapeDtypeStruct((2, 1, E), jnp.int32)),
)


def _atom_body(x_ref, d_ref, base_ref, h_ref):
    h_ref[...] = (jnp.dot(x_ref[...].astype(jnp.float32), d_ref[...],
                          preferred_element_type=jnp.float32,
                          precision=lax.Precision.HIGHEST)
                  + base_ref[...])


_atom = pl.pallas_call(
    _atom_body,
    out_shape=jax.ShapeDtypeStruct((N, EMB), jnp.float32),
)


BN = 2000
NB = N // BN


def _accum_stats(st_ref, y):
    i = pl.program_id(0)
    st = jnp.concatenate([jnp.sum(y, axis=0, keepdims=True),
                          jnp.sum(y * y, axis=0, keepdims=True)], axis=0)

    @pl.when(i == 0)
    def _():
        st_ref[...] = st

    @pl.when(i > 0)
    def _():
        st_ref[...] = st_ref[...] + st


def _mlp1_body(h_ref, a0_ref, a1_ref, sc_ref, W1_ref, b1_ref, y_ref, st_ref):
    agg = jnp.concatenate([a0_ref[...], a1_ref[...]], axis=1)
    z = sc_ref[0, 0] * h_ref[...] + agg
    y = jnp.dot(z, W1_ref[...], preferred_element_type=jnp.float32) + b1_ref[...]
    y_ref[...] = y
    _accum_stats(st_ref, y)


_mlp1 = pl.pallas_call(
    _mlp1_body,
    grid=(NB,),
    in_specs=[
        pl.BlockSpec((BN, EMB), lambda i: (i, 0)),
        pl.BlockSpec((BN, HALF), lambda i: (i, 0)),
        pl.BlockSpec((BN, HALF), lambda i: (i, 0)),
        pl.BlockSpec((1, 1), lambda i: (0, 0)),
        pl.BlockSpec((EMB, 2 * EMB), lambda i: (0, 0)),
        pl.BlockSpec((1, 2 * EMB), lambda i: (0, 0)),
    ],
    out_specs=(pl.BlockSpec((BN, 2 * EMB), lambda i: (i, 0)),
               pl.BlockSpec((2, 2 * EMB), lambda i: (0, 0))),
    out_shape=(jax.ShapeDtypeStruct((N, 2 * EMB), jnp.float32),
               jax.ShapeDtypeStruct((2, 2 * EMB), jnp.float32)),
)


def _mlp2_body(y_ref, st_ref, g1_ref, be1_ref, W2_ref, b2_ref,
               z_ref, st2_ref):
    mu = st_ref[0:1, :] * (1.0 / N)
    var = st_ref[1:2, :] * (1.0 / N) - mu * mu
    yn = (y_ref[...] - mu) / jnp.sqrt(var + 1e-5) * g1_ref[...] + be1_ref[...]
    yn = jnp.maximum(yn, 0.0)
    z2 = jnp.dot(yn, W2_ref[...], preferred_element_type=jnp.float32) + b2_ref[...]
    z_ref[...] = z2
    _accum_stats(st2_ref, z2)


_mlp2 = pl.pallas_call(
    _mlp2_body,
    grid=(NB,),
    in_specs=[
        pl.BlockSpec((BN, 2 * EMB), lambda i: (i, 0)),
        pl.BlockSpec((2, 2 * EMB), lambda i: (0, 0)),
        pl.BlockSpec((1, 2 * EMB), lambda i: (0, 0)),
        pl.BlockSpec((1, 2 * EMB), lambda i: (0, 0)),
        pl.BlockSpec((2 * EMB, EMB), lambda i: (0, 0)),
        pl.BlockSpec((1, EMB), lambda i: (0, 0)),
    ],
    out_specs=(pl.BlockSpec((BN, EMB), lambda i: (i, 0)),
               pl.BlockSpec((2, EMB), lambda i: (0, 0))),
    out_shape=(jax.ShapeDtypeStruct((N, EMB), jnp.float32),
               jax.ShapeDtypeStruct((2, EMB), jnp.float32)),
)


def _mlp3_body(z_ref, st2_ref, og_ref, ob_ref, out_ref, *, last):
    mu = st2_ref[0:1, :] * (1.0 / N)
    var = st2_ref[1:2, :] * (1.0 / N) - mu * mu
    o = (z_ref[...] - mu) / jnp.sqrt(var + 1e-5) * og_ref[...] + ob_ref[...]
    if not last:
        o = jnp.maximum(o, 0.0)
    out_ref[...] = o


def _make_mlp3(last):
    return pl.pallas_call(
        functools.partial(_mlp3_body, last=last),
        grid=(NB,),
        in_specs=[
            pl.BlockSpec((BN, EMB), lambda i: (i, 0)),
            pl.BlockSpec((2, EMB), lambda i: (0, 0)),
            pl.BlockSpec((1, EMB), lambda i: (0, 0)),
            pl.BlockSpec((1, EMB), lambda i: (0, 0)),
        ],
        out_specs=pl.BlockSpec((BN, EMB), lambda i: (i, 0)),
        out_shape=jax.ShapeDtypeStruct((N, EMB), jnp.float32),
    )


_mlp3_mid = _make_mlp3(False)
_mlp3_last = _make_mlp3(True)


def kernel(x, edge_index, edge_attr, atom_tables, bond0, bond1, bond2,
           W1, b1, g1, be1, W2, b2, eps, og, ob):
    f32 = jnp.float32
    d_atom = atom_tables[:, 1, :] - atom_tables[:, 0, :]
    base_atom = atom_tables[:, 0, :].sum(0)[None].astype(f32)
    d16 = jnp.zeros((16, EMB), f32).at[:9].set(d_atom)
    x16 = jnp.zeros((N, 16), jnp.int32).at[:, :9].set(x)
    q = jnp.arange(8)
    ee8 = bond0[:, q & 1] + bond1[:, (q >> 1) & 1] + bond2[:, (q >> 2) & 1]
    ee2 = ee8.reshape(L, 2 * 8, HALF)

    gidx, gcode = _prep(edge_index, edge_attr.T)
    dstl = edge_index[1]

    h = _atom(x16, d16, base_atom)
    for l in range(L):
        agg2 = _sc_msg(h.reshape(2 * N, HALF), gidx, gcode, dstl, ee2[l])
        sc = jnp.full((1, 1), 1.0, f32) + eps[l]
        y, st = _mlp1(h, agg2[0], agg2[1], sc, W1[l], b1[l][None])
        z2, st2 = _mlp2(y, st, g1[l][None], be1[l][None], W2[l], b2[l][None])
        mlp3 = _mlp3_last if l == L - 1 else _mlp3_mid
        h = mlp3(z2, st2, og[l][None], ob[l][None])
    return h
